# Initial kernel scaffold; baseline (speedup 1.0000x reference)
import functools

import jax
import jax.numpy as jnp
from jax import lax
from jax.experimental import pallas as pl
from jax.experimental.pallas import tpu as pltpu
from jax.experimental.pallas import tpu_sc as plsc

N = 10000
D = 128
NC = 2
NS = 16
NW = NC * NS
NP = 10240
RPT = NP // NS
E_BLK = 128

_mesh = plsc.VectorSubcoreMesh(core_axis_name="c", subcore_axis_name="s")


def _deg_body(dst_hbm, out_hbm, dst_v, ones_v, zb_v, deg_sh, nchunks):
    cid = lax.axis_index("c")
    sid = lax.axis_index("s")
    w = cid * NS + sid

    for i in range(RPT // 16):
        zb_v[pl.ds(i * 16, 16)] = jnp.zeros((16,), jnp.float32)
    pltpu.sync_copy(zb_v, deg_sh.at[pl.ds(sid * RPT, RPT)])
    for i in range(8):
        ones_v[pl.ds(i * 16, 16)] = jnp.ones((16,), jnp.float32)
    plsc.subcore_barrier()

    pltpu.sync_copy(dst_hbm.at[w], dst_v)

    def step(j, carry):
        pltpu.sync_copy(ones_v, deg_sh.at[dst_v.at[j]], add=True)
        return carry

    lax.fori_loop(0, nchunks, step, 0)
    plsc.subcore_barrier()
    pltpu.sync_copy(deg_sh.at[pl.ds(sid * RPT, RPT)],
                    out_hbm.at[cid, pl.ds(sid * RPT, RPT)])


def _scat_body(u_hbm, src_hbm, dst_hbm, out_hbm,
               src_v, dst_v, rows_v, y_sh, sem, nchunks):
    cid = lax.axis_index("c")
    sid = lax.axis_index("s")
    w = cid * NS + sid

    def zrow(i, carry):
        for c in range(D // 16):
            rows_v[i, pl.ds(c * 16, 16)] = jnp.zeros((16,), jnp.float32)
        return carry

    lax.fori_loop(0, E_BLK, zrow, 0)
    for k in range(RPT // E_BLK):
        pltpu.sync_copy(rows_v, y_sh.at[pl.ds(sid * RPT + k * E_BLK, E_BLK)])
    plsc.subcore_barrier()

    pltpu.sync_copy(src_hbm.at[w], src_v)
    pltpu.sync_copy(dst_hbm.at[w], dst_v)

    def step(j, carry):
        pltpu.async_copy(u_hbm.at[src_v.at[j]], rows_v, sem).wait()
        pltpu.sync_copy(rows_v, y_sh.at[dst_v.at[j]], add=True)
        return carry

    lax.fori_loop(0, nchunks, step, 0)
    plsc.subcore_barrier()
    pltpu.sync_copy(y_sh.at[pl.ds(sid * RPT, RPT)],
                    out_hbm.at[cid, pl.ds(sid * RPT, RPT)])


def _make_deg_kernel(nchunks):
    return functools.partial(
        pl.kernel,
        out_type=jax.ShapeDtypeStruct((NC, NP), jnp.float32),
        mesh=_mesh,
        scratch_types=[
            pltpu.VMEM((nchunks, E_BLK), jnp.int32),
            pltpu.VMEM((E_BLK,), jnp.float32),
            pltpu.VMEM((RPT,), jnp.float32),
            pltpu.VMEM_SHARED((NP,), jnp.float32),
        ],
    )(functools.partial(_deg_body, nchunks=nchunks))


def _make_scat_kernel(nchunks):
    return functools.partial(
        pl.kernel,
        out_type=jax.ShapeDtypeStruct((NC, NP, D), jnp.float32),
        mesh=_mesh,
        scratch_types=[
            pltpu.VMEM((nchunks, E_BLK), jnp.int32),
            pltpu.VMEM((nchunks, E_BLK), jnp.int32),
            pltpu.VMEM((E_BLK, D), jnp.float32),
            pltpu.VMEM_SHARED((NP, D), jnp.float32),
            pltpu.SemaphoreType.DMA,
        ],
    )(functools.partial(_scat_body, nchunks=nchunks))


M_BLK = 1024
GRID = NP // M_BLK


def _dinv(degs_ref):
    return lax.rsqrt(degs_ref[0] + degs_ref[1] + 1.0)


def _mm_scale_body(x_ref, w_ref, degs_ref, u_ref):
    xw = jnp.dot(x_ref[...], w_ref[...], preferred_element_type=jnp.float32)
    u_ref[...] = xw * _dinv(degs_ref)


def _mid_body(y_ref, u_ref, degs_ref, b_ref, w_ref, u2_ref):
    dinv = _dinv(degs_ref)
    t = (y_ref[0] + y_ref[1] + u_ref[...]) * dinv + b_ref[...]
    h = jnp.maximum(t, 0.0)
    u2_ref[...] = jnp.dot(h, w_ref[...], preferred_element_type=jnp.float32) * dinv


def _final_body(y_ref, u_ref, degs_ref, b_ref, out_ref):
    out_ref[...] = (y_ref[0] + y_ref[1] + u_ref[...]) * _dinv(degs_ref) + b_ref[...]


_row_spec = pl.BlockSpec((M_BLK, D), lambda i: (i, 0))
_w_spec = pl.BlockSpec((D, D), lambda i: (0, 0))
_deg_spec = pl.BlockSpec((NC, M_BLK, 1), lambda i: (0, i, 0))
_b_spec = pl.BlockSpec((1, D), lambda i: (0, 0))
_y_spec = pl.BlockSpec((NC, M_BLK, D), lambda i: (0, i, 0))
_row_out = jax.ShapeDtypeStruct((NP, D), jnp.float32)

_mm_scale = pl.pallas_call(
    _mm_scale_body, grid=(GRID,),
    in_specs=[_row_spec, _w_spec, _deg_spec],
    out_specs=_row_spec, out_shape=_row_out)

_mid = pl.pallas_call(
    _mid_body, grid=(GRID,),
    in_specs=[_y_spec, _row_spec, _deg_spec, _b_spec, _w_spec],
    out_specs=_row_spec, out_shape=_row_out)

_final = pl.pallas_call(
    _final_body, grid=(GRID,),
    in_specs=[_y_spec, _row_spec, _deg_spec, _b_spec],
    out_specs=_row_spec, out_shape=_row_out)


def kernel(x, edge_index, W1, b1, W2, b2):
    src = edge_index[0].astype(jnp.int32)
    dst = edge_index[1].astype(jnp.int32)
    e = src.shape[0]
    per_worker = E_BLK * NW
    nchunks = -(-e // per_worker)
    e_pad = nchunks * per_worker
    src_a = jnp.concatenate(
        [src, jnp.zeros((e_pad - e,), jnp.int32)]).reshape(NW, nchunks, E_BLK)
    dst_a = jnp.concatenate(
        [dst, jnp.full((e_pad - e,), N, jnp.int32)]).reshape(NW, nchunks, E_BLK)

    x_p = jnp.pad(x, ((0, NP - N), (0, 0)))
    b1r = b1.reshape(1, D)
    b2r = b2.reshape(1, D)

    degs = _make_deg_kernel(nchunks)(dst_a).reshape(NC, NP, 1)
    scat = _make_scat_kernel(nchunks)

    u1 = _mm_scale(x_p, W1, degs)
    y1 = scat(u1, src_a, dst_a)
    u2 = _mid(y1, u1, degs, b1r, W2)
    y2 = scat(u2, src_a, dst_a)
    out = _final(y2, u2, degs, b2r)
    return out[:N]


# speedup vs baseline: 13.6104x; 13.6104x over previous
import functools

import jax
import jax.numpy as jnp
from jax import lax
from jax.experimental import pallas as pl
from jax.experimental.pallas import tpu as pltpu
from jax.experimental.pallas import tpu_sc as plsc

N = 10000
D = 128
NC = 2
NS = 16
NW = NC * NS
NP = 10240
RPT = NP // NS
E_BLK = 128

@functools.cache
def _mesh():
    return plsc.VectorSubcoreMesh(
        core_axis_name="c", subcore_axis_name="s",
        num_cores=NC, num_subcores=NS)


def _deg_body(dst_hbm, out_hbm, dst_v, ones_v, zb_v, deg_sh, nchunks):
    cid = lax.axis_index("c")
    sid = lax.axis_index("s")
    w = cid * NS + sid

    for i in range(RPT // 16):
        zb_v[pl.ds(i * 16, 16)] = jnp.zeros((16,), jnp.float32)
    pltpu.sync_copy(zb_v, deg_sh.at[pl.ds(sid * RPT, RPT)])
    for i in range(8):
        ones_v[pl.ds(i * 16, 16)] = jnp.ones((16,), jnp.float32)
    plsc.subcore_barrier()

    pltpu.sync_copy(dst_hbm.at[w], dst_v)

    def step(j, carry):
        pltpu.sync_copy(ones_v, deg_sh.at[dst_v.at[j]], add=True)
        return carry

    lax.fori_loop(0, nchunks, step, 0)
    plsc.subcore_barrier()
    pltpu.sync_copy(deg_sh.at[pl.ds(sid * RPT, RPT)],
                    out_hbm.at[cid, pl.ds(sid * RPT, RPT)])


def _scat_body(u_hbm, src_hbm, dst_hbm, out_hbm,
               src_v, dst_v, rows_v, y_sh, sem, nchunks):
    cid = lax.axis_index("c")
    sid = lax.axis_index("s")
    w = cid * NS + sid

    def zrow(i, carry):
        for c in range(D // 16):
            rows_v[i, pl.ds(c * 16, 16)] = jnp.zeros((16,), jnp.float32)
        return carry

    lax.fori_loop(0, E_BLK, zrow, 0)
    for k in range(RPT // E_BLK):
        pltpu.sync_copy(rows_v, y_sh.at[pl.ds(sid * RPT + k * E_BLK, E_BLK)])
    plsc.subcore_barrier()

    pltpu.sync_copy(src_hbm.at[w], src_v)
    pltpu.sync_copy(dst_hbm.at[w], dst_v)

    def step(j, carry):
        pltpu.async_copy(u_hbm.at[src_v.at[j]], rows_v, sem).wait()
        pltpu.sync_copy(rows_v, y_sh.at[dst_v.at[j]], add=True)
        return carry

    lax.fori_loop(0, nchunks, step, 0)
    plsc.subcore_barrier()
    pltpu.sync_copy(y_sh.at[pl.ds(sid * RPT, RPT)],
                    out_hbm.at[cid, pl.ds(sid * RPT, RPT)])


def _make_deg_kernel(nchunks):
    return functools.partial(
        pl.kernel,
        out_type=jax.ShapeDtypeStruct((NC, NP), jnp.float32),
        mesh=_mesh(),
        scratch_types=[
            pltpu.VMEM((nchunks, E_BLK), jnp.int32),
            pltpu.VMEM((E_BLK,), jnp.float32),
            pltpu.VMEM((RPT,), jnp.float32),
            pltpu.VMEM_SHARED((NP,), jnp.float32),
        ],
    )(functools.partial(_deg_body, nchunks=nchunks))


def _make_scat_kernel(nchunks):
    return functools.partial(
        pl.kernel,
        out_type=jax.ShapeDtypeStruct((NC, NP, D), jnp.float32),
        mesh=_mesh(),
        scratch_types=[
            pltpu.VMEM((nchunks, E_BLK), jnp.int32),
            pltpu.VMEM((nchunks, E_BLK), jnp.int32),
            pltpu.VMEM((E_BLK, D), jnp.float32),
            pltpu.VMEM_SHARED((NP, D), jnp.float32),
            pltpu.SemaphoreType.DMA,
        ],
    )(functools.partial(_scat_body, nchunks=nchunks))


M_BLK = 1024
GRID = NP // M_BLK


def _dinv(degs_ref):
    return lax.rsqrt(degs_ref[0] + degs_ref[1] + 1.0)


def _mm_scale_body(x_ref, w_ref, degs_ref, u_ref):
    xw = jnp.dot(x_ref[...], w_ref[...], preferred_element_type=jnp.float32)
    u_ref[...] = xw * _dinv(degs_ref)


def _mid_body(y_ref, u_ref, degs_ref, b_ref, w_ref, u2_ref):
    dinv = _dinv(degs_ref)
    t = (y_ref[0] + y_ref[1] + u_ref[...]) * dinv + b_ref[...]
    h = jnp.maximum(t, 0.0)
    u2_ref[...] = jnp.dot(h, w_ref[...], preferred_element_type=jnp.float32) * dinv


def _final_body(y_ref, u_ref, degs_ref, b_ref, out_ref):
    out_ref[...] = (y_ref[0] + y_ref[1] + u_ref[...]) * _dinv(degs_ref) + b_ref[...]


_row_spec = pl.BlockSpec((M_BLK, D), lambda i: (i, 0))
_w_spec = pl.BlockSpec((D, D), lambda i: (0, 0))
_deg_spec = pl.BlockSpec((NC, M_BLK, 1), lambda i: (0, i, 0))
_b_spec = pl.BlockSpec((1, D), lambda i: (0, 0))
_y_spec = pl.BlockSpec((NC, M_BLK, D), lambda i: (0, i, 0))
_row_out = jax.ShapeDtypeStruct((NP, D), jnp.float32)

_mm_scale = pl.pallas_call(
    _mm_scale_body, grid=(GRID,),
    in_specs=[_row_spec, _w_spec, _deg_spec],
    out_specs=_row_spec, out_shape=_row_out)

_mid = pl.pallas_call(
    _mid_body, grid=(GRID,),
    in_specs=[_y_spec, _row_spec, _deg_spec, _b_spec, _w_spec],
    out_specs=_row_spec, out_shape=_row_out)

_final = pl.pallas_call(
    _final_body, grid=(GRID,),
    in_specs=[_y_spec, _row_spec, _deg_spec, _b_spec],
    out_specs=_row_spec, out_shape=_row_out)


def kernel(x, edge_index, W1, b1, W2, b2):
    src = edge_index[0].astype(jnp.int32)
    dst = edge_index[1].astype(jnp.int32)
    e = src.shape[0]
    per_worker = E_BLK * NW
    nchunks = -(-e // per_worker)
    e_pad = nchunks * per_worker
    src_a = jnp.concatenate(
        [src, jnp.zeros((e_pad - e,), jnp.int32)]).reshape(NW, nchunks, E_BLK)
    dst_a = jnp.concatenate(
        [dst, jnp.full((e_pad - e,), N, jnp.int32)]).reshape(NW, nchunks, E_BLK)

    x_p = jnp.pad(x, ((0, NP - N), (0, 0)))
    b1r = b1.reshape(1, D)
    b2r = b2.reshape(1, D)

    degs = _make_deg_kernel(nchunks)(dst_a).reshape(NC, NP, 1)
    scat = _make_scat_kernel(nchunks)

    u1 = _mm_scale(x_p, W1, degs)
    y1 = scat(u1, src_a, dst_a)
    u2 = _mid(y1, u1, degs, b1r, W2)
    y2 = scat(u2, src_a, dst_a)
    out = _final(y2, u2, degs, b2r)
    return out[:N]
